# baseline (device time: 125452 ns/iter reference)
import jax
import jax.numpy as jnp
from jax import lax
from jax.experimental import pallas as pl
from jax.experimental.pallas import tpu as pltpu

M = 2048
PAD = 8
M2 = M + PAD
NBITS = 9


def _convert_body(x_ref, o_ref):
    o_ref[:, :] = x_ref[:, :].astype(jnp.bfloat16)


def _convert(x):
    return pl.pallas_call(
        _convert_body,
        out_shape=jax.ShapeDtypeStruct(x.shape, jnp.bfloat16),
        in_specs=[pl.BlockSpec(memory_space=pltpu.VMEM)],
        out_specs=pl.BlockSpec(memory_space=pltpu.VMEM),
    )(x)


def _body(scal_ref, xs_ref, out_ref, send_sems, recv_sems, keep_sems):
    my_xi = lax.axis_index("x")
    my_yi = lax.axis_index("y")
    peer = (1 - my_xi, my_yi)
    dev0 = my_xi == 0

    k8 = scal_ref[0]
    k = scal_ref[1]
    n_keep = scal_ref[2]
    m8 = k8 * 8
    fk = (k >> 3) << 3

    barrier = pltpu.get_barrier_semaphore()
    pl.semaphore_signal(
        barrier, inc=1, device_id=peer, device_id_type=pl.DeviceIdType.MESH
    )
    pl.semaphore_wait(barrier, 1)

    dst_base = jnp.where(dev0, 0, M - m8)

    def rdma_for_bit(b):
        rows = 8 << b
        off = ((k8 >> (b + 1)) << (b + 1)) * 8
        return pltpu.make_async_remote_copy(
            src_ref=xs_ref.at[pl.ds(pl.multiple_of(off, 16), rows), :],
            dst_ref=out_ref.at[pl.ds(pl.multiple_of(dst_base + off, 8), rows), :],
            send_sem=send_sems.at[b],
            recv_sem=recv_sems.at[b],
            device_id=peer,
            device_id_type=pl.DeviceIdType.MESH,
        )

    for b in range(NBITS - 1, -1, -1):
        @pl.when(((k8 >> b) & 1) == 1)
        def _(b=b):
            rdma_for_bit(b).start()

    kb8 = jnp.where(dev0, 0, fk)
    w0 = jnp.where(dev0, 0, 1)
    nwin = jnp.where(dev0, n_keep >> 3, (M - fk) >> 3)
    q = nwin - w0

    def keep_for_bit(b):
        rows = 8 << b
        woff = w0 + ((q >> (b + 1)) << (b + 1))
        return pltpu.make_async_copy(
            xs_ref.at[pl.ds(pl.multiple_of(m8 + 8 * woff, 8), rows), :],
            out_ref.at[pl.ds(pl.multiple_of(kb8 + 8 * woff, 8), rows), :],
            keep_sems.at[b],
        )

    for b in range(NBITS - 1, -1, -1):
        @pl.when(((q >> b) & 1) == 1)
        def _(b=b):
            keep_for_bit(b).start()

    for b in range(NBITS - 1, -1, -1):
        @pl.when(((q >> b) & 1) == 1)
        def _(b=b):
            keep_for_bit(b).wait()

    for b in range(NBITS - 1, -1, -1):
        @pl.when(((k8 >> b) & 1) == 1)
        def _(b=b):
            rdma_for_bit(b).wait()

    wb = jnp.where(dev0, n_keep >> 3, 0)
    w8 = kb8 + 8 * wb
    r = jnp.where(dev0, n_keep & 7, k & 7)
    xs_win = xs_ref[pl.ds(pl.multiple_of(m8 + 8 * wb, 8), 8), :]
    out_win = out_ref[pl.ds(pl.multiple_of(w8, 8), 8), :]
    t = lax.broadcasted_iota(jnp.int32, (8, 1), 0)
    tl = (t < r).astype(jnp.int32)
    res = jnp.where(tl == dev0.astype(jnp.int32), xs_win, out_win)
    out_ref[pl.ds(pl.multiple_of(w8, 8), 8), :] = res


def kernel(x, dest):
    m, n = x.shape
    my_xi = lax.axis_index("x")
    dev0 = my_xi == 0

    xb = _convert(x)
    to_peer = (dest != my_xi).astype(jnp.int32)
    k = jnp.sum(to_peer)
    k8 = (k + 7) >> 3
    m8 = k8 * 8
    n_keep = m - k

    s0 = jnp.where(dev0, 0, m8 - k)
    ks = jnp.where(dev0, 0, k & 7)

    perm = jnp.argsort(1 - to_peer, stable=True)
    p = jnp.arange(M2, dtype=jnp.int32)
    rank = jnp.where(p < m8, p - s0, k + p - m8 - ks)
    g = perm[jnp.clip(rank, 0, m - 1)]
    xs = xb[g]

    scal = jnp.stack([k8, k, n_keep]).astype(jnp.int32)

    return pl.pallas_call(
        _body,
        out_shape=jax.ShapeDtypeStruct((m, n), jnp.bfloat16),
        in_specs=[
            pl.BlockSpec(memory_space=pltpu.SMEM),
            pl.BlockSpec(memory_space=pltpu.VMEM),
        ],
        out_specs=pl.BlockSpec(memory_space=pltpu.VMEM),
        scratch_shapes=[
            pltpu.SemaphoreType.DMA((NBITS,)),
            pltpu.SemaphoreType.DMA((NBITS,)),
            pltpu.SemaphoreType.DMA((NBITS,)),
        ],
        compiler_params=pltpu.CompilerParams(
            collective_id=0, vmem_limit_bytes=100 * 1024 * 1024
        ),
    )(scal, xs)


# device time: 62566 ns/iter; 2.0051x vs baseline; 2.0051x over previous
import jax
import jax.numpy as jnp
from jax import lax
from jax.experimental import pallas as pl
from jax.experimental.pallas import tpu as pltpu

M = 2048
PAD = 8
M2 = M + PAD
NBITS = 9


def _convert_body(x_ref, o_ref):
    o_ref[:, :] = x_ref[:, :].astype(jnp.bfloat16)


def _convert(x):
    return pl.pallas_call(
        _convert_body,
        out_shape=jax.ShapeDtypeStruct(x.shape, jnp.bfloat16),
        in_specs=[pl.BlockSpec(memory_space=pltpu.VMEM)],
        out_specs=pl.BlockSpec(memory_space=pltpu.VMEM),
    )(x)


def _body(scal_ref, xs_ref, out_ref, send_sems, recv_sems, keep_sems):
    my_xi = lax.axis_index("x")
    my_yi = lax.axis_index("y")
    peer = (1 - my_xi, my_yi)
    dev0 = my_xi == 0

    k8 = scal_ref[0]
    k = scal_ref[1]
    n_keep = scal_ref[2]
    m8 = k8 * 8
    fk = (k >> 3) << 3

    barrier = pltpu.get_barrier_semaphore()
    pl.semaphore_signal(
        barrier, inc=1, device_id=peer, device_id_type=pl.DeviceIdType.MESH
    )
    pl.semaphore_wait(barrier, 1)

    dst_base = jnp.where(dev0, 0, M - m8)

    def rdma_for_bit(b):
        rows = 8 << b
        off = ((k8 >> (b + 1)) << (b + 1)) * 8
        return pltpu.make_async_remote_copy(
            src_ref=xs_ref.at[pl.ds(pl.multiple_of(off, 16), rows), :],
            dst_ref=out_ref.at[pl.ds(pl.multiple_of(dst_base + off, 8), rows), :],
            send_sem=send_sems.at[b],
            recv_sem=recv_sems.at[b],
            device_id=peer,
            device_id_type=pl.DeviceIdType.MESH,
        )

    for b in range(NBITS - 1, -1, -1):
        @pl.when(((k8 >> b) & 1) == 1)
        def _(b=b):
            rdma_for_bit(b).start()

    kb8 = jnp.where(dev0, 0, fk)
    w0 = jnp.where(dev0, 0, 1)
    nwin = jnp.where(dev0, n_keep >> 3, (M - fk) >> 3)
    q = nwin - w0

    def keep_for_bit(b):
        rows = 8 << b
        woff = w0 + ((q >> (b + 1)) << (b + 1))
        return pltpu.make_async_copy(
            xs_ref.at[pl.ds(pl.multiple_of(m8 + 8 * woff, 8), rows), :],
            out_ref.at[pl.ds(pl.multiple_of(kb8 + 8 * woff, 8), rows), :],
            keep_sems.at[b],
        )

    for b in range(NBITS - 1, -1, -1):
        @pl.when(((q >> b) & 1) == 1)
        def _(b=b):
            keep_for_bit(b).start()

    for b in range(NBITS - 1, -1, -1):
        @pl.when(((q >> b) & 1) == 1)
        def _(b=b):
            keep_for_bit(b).wait()

    for b in range(NBITS - 1, -1, -1):
        @pl.when(((k8 >> b) & 1) == 1)
        def _(b=b):
            rdma_for_bit(b).wait()

    wb = jnp.where(dev0, n_keep >> 3, 0)
    w8 = kb8 + 8 * wb
    r = jnp.where(dev0, n_keep & 7, k & 7)
    xs_win = xs_ref[pl.ds(pl.multiple_of(m8 + 8 * wb, 8), 8), :]
    out_win = out_ref[pl.ds(pl.multiple_of(w8, 8), 8), :]
    t = lax.broadcasted_iota(jnp.int32, (8, 1), 0)
    tl = (t < r).astype(jnp.int32)
    res = jnp.where(tl == dev0.astype(jnp.int32), xs_win, out_win)
    out_ref[pl.ds(pl.multiple_of(w8, 8), 8), :] = res


def kernel(x, dest):
    m, n = x.shape
    my_xi = lax.axis_index("x")
    dev0 = my_xi == 0

    xb = _convert(x)
    to_peer = (dest != my_xi).astype(jnp.int32)
    k = jnp.sum(to_peer)
    k8 = (k + 7) >> 3
    m8 = k8 * 8
    n_keep = m - k

    s0 = jnp.where(dev0, 0, m8 - k)
    ks = jnp.where(dev0, 0, k & 7)

    keep_f = 1 - to_peer
    excl_peer = jnp.cumsum(to_peer) - to_peer
    excl_keep = jnp.cumsum(keep_f) - keep_f
    d = jnp.where(to_peer == 1, s0 + excl_peer, m8 + ks + excl_keep)
    g = jnp.zeros((M2,), jnp.int32).at[d].set(jnp.arange(m, dtype=jnp.int32))
    xs = xb[g]

    scal = jnp.stack([k8, k, n_keep]).astype(jnp.int32)

    return pl.pallas_call(
        _body,
        out_shape=jax.ShapeDtypeStruct((m, n), jnp.bfloat16),
        in_specs=[
            pl.BlockSpec(memory_space=pltpu.SMEM),
            pl.BlockSpec(memory_space=pltpu.VMEM),
        ],
        out_specs=pl.BlockSpec(memory_space=pltpu.VMEM),
        scratch_shapes=[
            pltpu.SemaphoreType.DMA((NBITS,)),
            pltpu.SemaphoreType.DMA((NBITS,)),
            pltpu.SemaphoreType.DMA((NBITS,)),
        ],
        compiler_params=pltpu.CompilerParams(
            collective_id=0, vmem_limit_bytes=100 * 1024 * 1024
        ),
    )(scal, xs)


# device time: 55152 ns/iter; 2.2747x vs baseline; 1.1344x over previous
import jax
import jax.numpy as jnp
from jax import lax
from jax.experimental import pallas as pl
from jax.experimental.pallas import tpu as pltpu

M = 2048
PAD = 16
M2 = M + PAD
NBITS = 9


def _convert_body(x_ref, o_ref):
    o_ref[:, :] = x_ref[:, :].astype(jnp.bfloat16)


def _convert(x):
    return pl.pallas_call(
        _convert_body,
        out_shape=jax.ShapeDtypeStruct(x.shape, jnp.bfloat16),
        in_specs=[pl.BlockSpec(memory_space=pltpu.VMEM)],
        out_specs=pl.BlockSpec(memory_space=pltpu.VMEM),
    )(x)


def _body(scal_ref, xs_ref, out_ref, send_sems, recv_sems, keep_sems):
    my_xi = lax.axis_index("x")
    my_yi = lax.axis_index("y")
    peer = (1 - my_xi, my_yi)
    dev0 = my_xi == 0

    k8 = scal_ref[0]
    k = scal_ref[1]
    n_keep = scal_ref[2]
    m8 = k8 * 8
    fk = (k >> 3) << 3

    barrier = pltpu.get_barrier_semaphore()
    pl.semaphore_signal(
        barrier, inc=1, device_id=peer, device_id_type=pl.DeviceIdType.MESH
    )
    pl.semaphore_wait(barrier, 1)

    dst_base = jnp.where(dev0, 0, M - m8)

    def rdma_for_bit(b):
        rows = 8 << b
        off = ((k8 >> (b + 1)) << (b + 1)) * 8
        return pltpu.make_async_remote_copy(
            src_ref=xs_ref.at[pl.ds(pl.multiple_of(off, 16), rows), :],
            dst_ref=out_ref.at[pl.ds(pl.multiple_of(dst_base + off, 8), rows), :],
            send_sem=send_sems.at[b],
            recv_sem=recv_sems.at[b],
            device_id=peer,
            device_id_type=pl.DeviceIdType.MESH,
        )

    for b in range(NBITS - 1, -1, -1):
        @pl.when(((k8 >> b) & 1) == 1)
        def _(b=b):
            rdma_for_bit(b).start()

    kb8 = jnp.where(dev0, 0, fk)
    w0 = jnp.where(dev0, 0, 1)
    nwin = jnp.where(dev0, n_keep >> 3, (M - fk) >> 3)
    q = nwin - w0

    def keep_for_bit(b):
        rows = 8 << b
        woff = w0 + ((q >> (b + 1)) << (b + 1))
        return pltpu.make_async_copy(
            xs_ref.at[pl.ds(pl.multiple_of(m8 + 8 * woff, 8), rows), :],
            out_ref.at[pl.ds(pl.multiple_of(kb8 + 8 * woff, 8), rows), :],
            keep_sems.at[b],
        )

    for b in range(NBITS - 1, -1, -1):
        @pl.when(((q >> b) & 1) == 1)
        def _(b=b):
            keep_for_bit(b).start()

    for b in range(NBITS - 1, -1, -1):
        @pl.when(((q >> b) & 1) == 1)
        def _(b=b):
            keep_for_bit(b).wait()

    for b in range(NBITS - 1, -1, -1):
        @pl.when(((k8 >> b) & 1) == 1)
        def _(b=b):
            rdma_for_bit(b).wait()

    wb = jnp.where(dev0, n_keep >> 3, 0)
    w8 = kb8 + 8 * wb
    r = jnp.where(dev0, n_keep & 7, k & 7)
    xs_win = xs_ref[pl.ds(pl.multiple_of(m8 + 8 * wb, 8), 8), :]
    out_win = out_ref[pl.ds(pl.multiple_of(w8, 8), 8), :]
    t = lax.broadcasted_iota(jnp.int32, (8, 1), 0)
    tl = (t < r).astype(jnp.int32)
    res = jnp.where(tl == dev0.astype(jnp.int32), xs_win, out_win)
    out_ref[pl.ds(pl.multiple_of(w8, 8), 8), :] = res


def kernel(x, dest):
    m, n = x.shape
    my_xi = lax.axis_index("x")
    dev0 = my_xi == 0

    xb = _convert(x)
    to_peer = (dest != my_xi).astype(jnp.int32)
    k = jnp.sum(to_peer)
    k8 = (k + 7) >> 3
    m8 = k8 * 8
    n_keep = m - k

    s0 = jnp.where(dev0, 0, m8 - k)
    ks = jnp.where(dev0, 0, k & 7)

    keep_f = 1 - to_peer
    excl_peer = jnp.cumsum(to_peer) - to_peer
    excl_keep = jnp.cumsum(keep_f) - keep_f
    d = jnp.where(to_peer == 1, s0 + excl_peer, m8 + ks + excl_keep)
    xs = jnp.zeros((M2, n), jnp.bfloat16).at[d].set(xb)

    scal = jnp.stack([k8, k, n_keep]).astype(jnp.int32)

    return pl.pallas_call(
        _body,
        out_shape=jax.ShapeDtypeStruct((m, n), jnp.bfloat16),
        in_specs=[
            pl.BlockSpec(memory_space=pltpu.SMEM),
            pl.BlockSpec(memory_space=pltpu.VMEM),
        ],
        out_specs=pl.BlockSpec(memory_space=pltpu.VMEM),
        scratch_shapes=[
            pltpu.SemaphoreType.DMA((NBITS,)),
            pltpu.SemaphoreType.DMA((NBITS,)),
            pltpu.SemaphoreType.DMA((NBITS,)),
        ],
        compiler_params=pltpu.CompilerParams(
            collective_id=0, vmem_limit_bytes=100 * 1024 * 1024
        ),
    )(scal, xs)
